# baseline (speedup 1.0000x reference)
import functools

import jax
import jax.numpy as jnp
from jax.experimental import pallas as pl
from jax.experimental.pallas import tpu as pltpu

N1 = 50000
N2 = 5000
UNITS = 128
NUM_CLASSES = 25
NUM_LAYERS = 7
EPS = 1e-5


def _bn(x, g, b):
    m = jnp.mean(x, axis=0)
    v = jnp.var(x, axis=0)
    return (x - m) * jax.lax.rsqrt(v + EPS) * g + b


def _spmm(edge_index, x, n_out):
    return jax.ops.segment_sum(x[edge_index[1]], edge_index[0], num_segments=n_out)


def _softmax_head_body(x_ref, w_ref, b_ref, o_ref):
    z = jnp.dot(x_ref[...], w_ref[...], preferred_element_type=jnp.float32)
    z = z + b_ref[...]
    m = jnp.max(z, axis=1, keepdims=True)
    e = jnp.exp(z - m)
    s = jnp.sum(e, axis=1, keepdims=True)
    o_ref[...] = e / s


def _softmax_head(x, w, b):
    n, _ = x.shape
    blk = 2000
    w_pad = jnp.zeros((UNITS, UNITS), jnp.float32).at[:, :NUM_CLASSES].set(w)
    b_pad = jnp.full((1, UNITS), -1e30, jnp.float32).at[0, :NUM_CLASSES].set(b)
    out = pl.pallas_call(
        _softmax_head_body,
        grid=(n // blk,),
        in_specs=[
            pl.BlockSpec((blk, UNITS), lambda i: (i, 0)),
            pl.BlockSpec((UNITS, UNITS), lambda i: (0, 0)),
            pl.BlockSpec((1, UNITS), lambda i: (0, 0)),
        ],
        out_specs=pl.BlockSpec((blk, UNITS), lambda i: (i, 0)),
        out_shape=jax.ShapeDtypeStruct((n, UNITS), jnp.float32),
    )(x, w_pad, b_pad)
    return out[:, :NUM_CLASSES]


def kernel(V_1, E_1, E_2, E_3, V_2, A_2, A_3, W_ge_start, b_ge_start, W0_1, W1_1, W2_1, W3_1, b_1, WI_2a, WA_2a, b_2a, WI_2, WA_2, b_2, W_ge1, b_ge1, W_ge2, b_ge2, W_a3, b_a3, W_a4, b_a4, W_gef, b_gef, bn_gamma, bn_beta, bn_a4_gamma, bn_a4_beta):
    relu = jax.nn.relu
    x1 = relu(_bn(V_1 @ W_ge_start + b_ge_start, bn_gamma[0], bn_beta[0]))
    a4 = jax.ops.segment_sum(x1, A_3, num_segments=N2) @ W_a4 + b_a4
    a4 = relu(_bn(a4, bn_a4_gamma, bn_a4_beta))
    x2 = V_2 + a4
    for i in range(1, NUM_LAYERS + 1):
        conv = _spmm(A_2, x2, N2)
        if i == 1:
            r = conv @ WA_2a + x2 @ WI_2a + b_2a
        else:
            r = conv @ WA_2[i - 2] + x2 @ WI_2[i - 2] + b_2[i - 2]
        r = relu(_bn(r, bn_gamma[7 + i], bn_beta[7 + i]))
        x2 = r if i == 1 else x2 + r
    x2 = relu(_bn(x2 @ W_ge2 + b_ge2, bn_gamma[16], bn_beta[16]))
    a3 = x2[A_3] @ W_a3 + b_a3
    a3 = relu(_bn(a3, bn_gamma[17], bn_beta[17]))
    x1 = x1 + a3
    for i in range(1, NUM_LAYERS + 1):
        r = (_spmm(E_1, x1, N1) @ W1_1[i - 1] + _spmm(E_2, x1, N1) @ W2_1[i - 1]
             + _spmm(E_3, x1, N1) @ W3_1[i - 1] + x1 @ W0_1[i - 1] + b_1[i - 1])
        r = relu(_bn(r, bn_gamma[i], bn_beta[i]))
        x1 = x1 + r
    x1 = relu(_bn(x1 @ W_ge1 + b_ge1, bn_gamma[15], bn_beta[15]))
    return _softmax_head(x1, W_gef, b_gef)


# speedup vs baseline: 1.8345x; 1.8345x over previous
import functools

import jax
import jax.numpy as jnp
from jax import lax
from jax.experimental import pallas as pl
from jax.experimental.pallas import tpu as pltpu
from jax.experimental.pallas import tpu_sc as plsc

N1 = 50000
N2 = 5000
UNITS = 128
NUM_CLASSES = 25
NUM_LAYERS = 7
EPS = 1e-5

NTILES = 16
NP1 = 50048
RPT1 = NP1 // 16
ZB1 = 136
NP2 = 5120
RPT2 = NP2 // 16
ZB2 = 64


def _bn(x, g, b):
    m = jnp.mean(x, axis=0)
    v = jnp.var(x, axis=0)
    return (x - m) * jax.lax.rsqrt(v + EPS) * g + b


@functools.cache
def _make_spmm(n_in_p, nblk, n_out_p, rpt, zb):
    nz = rpt // zb
    grp = 8
    assert rpt % zb == 0 and zb % 8 == 0 and rpt % 8 == 0 and nblk % grp == 0
    mesh = plsc.VectorSubcoreMesh(core_axis_name="c", subcore_axis_name="s")

    @functools.partial(
        pl.kernel,
        out_type=jax.ShapeDtypeStruct((4, n_out_p, 32), jnp.float32),
        mesh=mesh,
        compiler_params=pltpu.CompilerParams(use_tc_tiling_on_sc=False),
        scratch_types=[
            pltpu.VMEM((grp, 128), jnp.int32),
            pltpu.VMEM((grp, 128), jnp.int32),
            pltpu.VMEM((128, 32), jnp.float32),
            pltpu.VMEM((zb, 32), jnp.float32),
            pltpu.VMEM_SHARED((n_out_p + 8, 32), jnp.float32),
            pltpu.SemaphoreType.DMA,
        ],
    )
    def spmm(x_hbm, sidx_hbm, didx_hbm, out_hbm,
             sidx_v, didx_v, buf_v, zero_v, acc_sh, sem):
        c = lax.axis_index("c")
        s = lax.axis_index("s")

        zv = jnp.zeros((16,), jnp.float32)

        def zinit(i, carry):
            zero_v[i, pl.ds(0, 16)] = zv
            zero_v[i, pl.ds(16, 16)] = zv
            return carry

        lax.fori_loop(0, zb, zinit, 0)

        for k in range(2):
            p = c * 2 + k

            def zfill(i, carry):
                pltpu.sync_copy(zero_v, acc_sh.at[pl.ds(s * rpt + i * zb, zb)])
                return carry

            lax.fori_loop(0, nz, zfill, 0)
            plsc.subcore_barrier()

            def blk(g, carry):
                pltpu.sync_copy(sidx_hbm.at[p, s, pl.ds(g * grp, grp)], sidx_v)
                pltpu.sync_copy(didx_hbm.at[s, pl.ds(g * grp, grp)], didx_v)
                for jj in range(grp):
                    pltpu.async_copy(x_hbm.at[sidx_v.at[jj]], buf_v, sem).wait()
                    pltpu.sync_copy(buf_v, acc_sh.at[didx_v.at[jj]], add=True)
                return carry

            lax.fori_loop(0, nblk // grp, blk, 0)
            plsc.subcore_barrier()
            pltpu.sync_copy(
                acc_sh.at[pl.ds(s * rpt, rpt)],
                out_hbm.at[p, pl.ds(s * rpt, rpt)],
            )
            plsc.subcore_barrier()

    return spmm


def _pad_edges(src, dst, n_in_p, n_out_p):
    e = src.shape[0]
    per = -(-e // NTILES)
    nblk = -(-per // 128)
    nblk = -(-nblk // 8) * 8
    total = NTILES * nblk * 128
    pad = total - e
    srcp = jnp.concatenate([src.astype(jnp.int32), jnp.zeros((pad,), jnp.int32)])
    dstp = jnp.concatenate([dst.astype(jnp.int32),
                            jnp.full((pad,), n_out_p, jnp.int32)])
    srcp = srcp.reshape(1, NTILES, nblk, 128)
    offs = (jnp.arange(4, dtype=jnp.int32) * n_in_p).reshape(4, 1, 1, 1)
    return srcp + offs, dstp.reshape(NTILES, nblk, 128), nblk


def _sc_spmm(xc, src, dst, n_out_p, rpt, zb):
    n_in_p = xc.shape[1]
    sidx, didx, nblk = _pad_edges(src, dst, n_in_p, n_out_p)
    fn = _make_spmm(n_in_p, nblk, n_out_p, rpt, zb)
    return fn(xc.reshape(4 * n_in_p, 32), sidx, didx)


def _to_chunk(x, n_p):
    n = x.shape[0]
    xp = jnp.pad(x, ((0, n_p - n), (0, 0)))
    return xp.reshape(n_p, 4, 32).transpose(1, 0, 2)


def _from_chunk(xc, n):
    return xc.transpose(1, 0, 2).reshape(xc.shape[1], 128)[:n]


def _softmax_head_body(x_ref, w_ref, b_ref, o_ref):
    z = jnp.dot(x_ref[...], w_ref[...], preferred_element_type=jnp.float32)
    z = z + b_ref[...]
    m = jnp.max(z, axis=1, keepdims=True)
    e = jnp.exp(z - m)
    s = jnp.sum(e, axis=1, keepdims=True)
    o_ref[...] = e / s


def _softmax_head(x, w, b):
    n, _ = x.shape
    blk = 2000
    w_pad = jnp.zeros((UNITS, UNITS), jnp.float32).at[:, :NUM_CLASSES].set(w)
    b_pad = jnp.full((1, UNITS), -1e30, jnp.float32).at[0, :NUM_CLASSES].set(b)
    out = pl.pallas_call(
        _softmax_head_body,
        grid=(n // blk,),
        in_specs=[
            pl.BlockSpec((blk, UNITS), lambda i: (i, 0)),
            pl.BlockSpec((UNITS, UNITS), lambda i: (0, 0)),
            pl.BlockSpec((1, UNITS), lambda i: (0, 0)),
        ],
        out_specs=pl.BlockSpec((blk, UNITS), lambda i: (i, 0)),
        out_shape=jax.ShapeDtypeStruct((n, UNITS), jnp.float32),
    )(x, w_pad, b_pad)
    return out[:, :NUM_CLASSES]


def kernel(V_1, E_1, E_2, E_3, V_2, A_2, A_3, W_ge_start, b_ge_start, W0_1, W1_1, W2_1, W3_1, b_1, WI_2a, WA_2a, b_2a, WI_2, WA_2, b_2, W_ge1, b_ge1, W_ge2, b_ge2, W_a3, b_a3, W_a4, b_a4, W_gef, b_gef, bn_gamma, bn_beta, bn_a4_gamma, bn_a4_beta):
    relu = jax.nn.relu
    iota1 = jnp.arange(N1, dtype=jnp.int32)
    A_3 = A_3.astype(jnp.int32)

    x1 = relu(_bn(V_1 @ W_ge_start + b_ge_start, bn_gamma[0], bn_beta[0]))

    sA = _from_chunk(_sc_spmm(_to_chunk(x1, NP1), iota1, A_3, NP2, RPT2, ZB2), N2)
    a4 = sA @ W_a4 + b_a4
    a4 = relu(_bn(a4, bn_a4_gamma, bn_a4_beta))
    x2 = V_2 + a4

    x2p = jnp.pad(x2, ((0, 0), (0, UNITS - 4)))
    for i in range(1, NUM_LAYERS + 1):
        conv = _from_chunk(
            _sc_spmm(_to_chunk(x2p, NP2), A_2[1], A_2[0], NP2, RPT2, ZB2), N2)
        if i == 1:
            r = conv[:, :4] @ WA_2a + x2p[:, :4] @ WI_2a + b_2a
        else:
            r = conv @ WA_2[i - 2] + x2p @ WI_2[i - 2] + b_2[i - 2]
        r = relu(_bn(r, bn_gamma[7 + i], bn_beta[7 + i]))
        x2p = r if i == 1 else x2p + r
    x2f = relu(_bn(x2p @ W_ge2 + b_ge2, bn_gamma[16], bn_beta[16]))

    s3 = _from_chunk(
        _sc_spmm(_to_chunk(x2f, NP2), A_3, iota1, NP1, RPT1, ZB1), N1)
    a3 = s3 @ W_a3 + b_a3
    a3 = relu(_bn(a3, bn_gamma[17], bn_beta[17]))
    x1 = x1 + a3

    for i in range(1, NUM_LAYERS + 1):
        x1c = _to_chunk(x1, NP1)
        c1 = _from_chunk(_sc_spmm(x1c, E_1[1], E_1[0], NP1, RPT1, ZB1), N1)
        c2 = _from_chunk(_sc_spmm(x1c, E_2[1], E_2[0], NP1, RPT1, ZB1), N1)
        c3 = _from_chunk(_sc_spmm(x1c, E_3[1], E_3[0], NP1, RPT1, ZB1), N1)
        r = (c1 @ W1_1[i - 1] + c2 @ W2_1[i - 1] + c3 @ W3_1[i - 1]
             + x1 @ W0_1[i - 1] + b_1[i - 1])
        r = relu(_bn(r, bn_gamma[i], bn_beta[i]))
        x1 = x1 + r
    x1 = relu(_bn(x1 @ W_ge1 + b_ge1, bn_gamma[15], bn_beta[15]))
    return _softmax_head(x1, W_gef, b_gef)


# speedup vs baseline: 2.6616x; 1.4508x over previous
import functools

import jax
import jax.numpy as jnp
from jax import lax
from jax.experimental import pallas as pl
from jax.experimental.pallas import tpu as pltpu
from jax.experimental.pallas import tpu_sc as plsc

N1 = 50000
N2 = 5000
UNITS = 128
NUM_CLASSES = 25
NUM_LAYERS = 7
EPS = 1e-5

NTILES = 16
NP1 = 50048
RPT1 = NP1 // 16
ZB1 = 136
NP2 = 5120
RPT2 = NP2 // 16
ZB2 = 64


def _bn(x, g, b):
    m = jnp.mean(x, axis=0)
    v = jnp.var(x, axis=0)
    return (x - m) * jax.lax.rsqrt(v + EPS) * g + b


@functools.cache
def _make_spmm(n_in_p, nblk, n_out_p, rpt, zb):
    nz = rpt // zb
    half = 3
    nseg = nblk // (2 * half)
    assert rpt % zb == 0 and zb % 8 == 0 and rpt % 8 == 0
    assert nblk % (2 * half) == 0
    mesh = plsc.VectorSubcoreMesh(core_axis_name="c", subcore_axis_name="s")

    @functools.partial(
        pl.kernel,
        out_type=jax.ShapeDtypeStruct((4, n_out_p, 32), jnp.float32),
        mesh=mesh,
        compiler_params=pltpu.CompilerParams(use_tc_tiling_on_sc=False),
        scratch_types=[
            pltpu.VMEM((half, 128), jnp.int32),
            pltpu.VMEM((half, 128), jnp.int32),
            pltpu.VMEM((half, 128), jnp.int32),
            pltpu.VMEM((half, 128), jnp.int32),
            pltpu.VMEM((2 * half, 128, 32), jnp.float32),
            pltpu.VMEM((zb, 32), jnp.float32),
            pltpu.VMEM_SHARED((n_out_p + 8, 32), jnp.float32),
            pltpu.SemaphoreType.DMA,
            pltpu.SemaphoreType.DMA,
        ],
    )
    def spmm(x_hbm, sidx_hbm, didx_hbm, out_hbm,
             sidxa, didxa, sidxb, didxb, buf_v, zero_v, acc_sh, sema, semb):
        c = lax.axis_index("c")
        s = lax.axis_index("s")

        zv = jnp.zeros((16,), jnp.float32)

        def zinit(i, carry):
            zero_v[i, pl.ds(0, 16)] = zv
            zero_v[i, pl.ds(16, 16)] = zv
            return carry

        lax.fori_loop(0, zb, zinit, 0)

        for k in range(2):
            p = c * 2 + k

            def zfill(i, carry):
                pltpu.sync_copy(zero_v, acc_sh.at[pl.ds(s * rpt + i * zb, zb)])
                return carry

            lax.fori_loop(0, nz, zfill, 0)
            plsc.subcore_barrier()

            def fire(g, off, sidx_v, didx_v, sem):
                pltpu.sync_copy(
                    sidx_hbm.at[p, s, pl.ds(g * 2 * half + off, half)], sidx_v)
                pltpu.sync_copy(
                    didx_hbm.at[s, pl.ds(g * 2 * half + off, half)], didx_v)
                for jj in range(half):
                    pltpu.async_copy(
                        x_hbm.at[sidx_v.at[jj]], buf_v.at[off + jj], sem)

            def drain_scatter(off, didx_v, sem):
                for jj in range(half):
                    pltpu.make_async_copy(
                        x_hbm.at[pl.ds(0, 128)], buf_v.at[off + jj], sem
                    ).wait()
                for jj in range(half):
                    pltpu.sync_copy(
                        buf_v.at[off + jj], acc_sh.at[didx_v.at[jj]], add=True)

            fire(0, 0, sidxa, didxa, sema)

            def seg(g, carry):
                fire(g, half, sidxb, didxb, semb)
                drain_scatter(0, didxa, sema)

                @pl.when(g < nseg - 1)
                def _():
                    fire(g + 1, 0, sidxa, didxa, sema)

                drain_scatter(half, didxb, semb)
                return carry

            lax.fori_loop(0, nseg, seg, 0)
            plsc.subcore_barrier()
            pltpu.sync_copy(
                acc_sh.at[pl.ds(s * rpt, rpt)],
                out_hbm.at[p, pl.ds(s * rpt, rpt)],
            )
            plsc.subcore_barrier()

    return spmm


def _pad_edges(src, dst, n_in_p, n_out_p):
    e = src.shape[0]
    per = -(-e // NTILES)
    nblk = -(-per // 128)
    nblk = -(-nblk // 6) * 6
    total = NTILES * nblk * 128
    pad = total - e
    srcp = jnp.concatenate([src.astype(jnp.int32), jnp.zeros((pad,), jnp.int32)])
    dstp = jnp.concatenate([dst.astype(jnp.int32),
                            jnp.full((pad,), n_out_p, jnp.int32)])
    srcp = srcp.reshape(1, NTILES, nblk, 128)
    offs = (jnp.arange(4, dtype=jnp.int32) * n_in_p).reshape(4, 1, 1, 1)
    return srcp + offs, dstp.reshape(NTILES, nblk, 128), nblk


def _sc_spmm(xc, src, dst, n_out_p, rpt, zb):
    n_in_p = xc.shape[1]
    sidx, didx, nblk = _pad_edges(src, dst, n_in_p, n_out_p)
    fn = _make_spmm(n_in_p, nblk, n_out_p, rpt, zb)
    return fn(xc.reshape(4 * n_in_p, 32), sidx, didx)


def _to_chunk(x, n_p):
    n = x.shape[0]
    xp = jnp.pad(x, ((0, n_p - n), (0, 0)))
    return xp.reshape(n_p, 4, 32).transpose(1, 0, 2)


def _from_chunk(xc, n):
    return xc.transpose(1, 0, 2).reshape(xc.shape[1], 128)[:n]


def _softmax_head_body(x_ref, w_ref, b_ref, o_ref):
    z = jnp.dot(x_ref[...], w_ref[...], preferred_element_type=jnp.float32)
    z = z + b_ref[...]
    m = jnp.max(z, axis=1, keepdims=True)
    e = jnp.exp(z - m)
    s = jnp.sum(e, axis=1, keepdims=True)
    o_ref[...] = e / s


def _softmax_head(x, w, b):
    n, _ = x.shape
    blk = 2000
    w_pad = jnp.zeros((UNITS, UNITS), jnp.float32).at[:, :NUM_CLASSES].set(w)
    b_pad = jnp.full((1, UNITS), -1e30, jnp.float32).at[0, :NUM_CLASSES].set(b)
    out = pl.pallas_call(
        _softmax_head_body,
        grid=(n // blk,),
        in_specs=[
            pl.BlockSpec((blk, UNITS), lambda i: (i, 0)),
            pl.BlockSpec((UNITS, UNITS), lambda i: (0, 0)),
            pl.BlockSpec((1, UNITS), lambda i: (0, 0)),
        ],
        out_specs=pl.BlockSpec((blk, UNITS), lambda i: (i, 0)),
        out_shape=jax.ShapeDtypeStruct((n, UNITS), jnp.float32),
    )(x, w_pad, b_pad)
    return out[:, :NUM_CLASSES]


def kernel(V_1, E_1, E_2, E_3, V_2, A_2, A_3, W_ge_start, b_ge_start, W0_1, W1_1, W2_1, W3_1, b_1, WI_2a, WA_2a, b_2a, WI_2, WA_2, b_2, W_ge1, b_ge1, W_ge2, b_ge2, W_a3, b_a3, W_a4, b_a4, W_gef, b_gef, bn_gamma, bn_beta, bn_a4_gamma, bn_a4_beta):
    relu = jax.nn.relu
    iota1 = jnp.arange(N1, dtype=jnp.int32)
    A_3 = A_3.astype(jnp.int32)

    x1 = relu(_bn(V_1 @ W_ge_start + b_ge_start, bn_gamma[0], bn_beta[0]))

    sA = _from_chunk(_sc_spmm(_to_chunk(x1, NP1), iota1, A_3, NP2, RPT2, ZB2), N2)
    a4 = sA @ W_a4 + b_a4
    a4 = relu(_bn(a4, bn_a4_gamma, bn_a4_beta))
    x2 = V_2 + a4

    x2p = jnp.pad(x2, ((0, 0), (0, UNITS - 4)))
    for i in range(1, NUM_LAYERS + 1):
        conv = _from_chunk(
            _sc_spmm(_to_chunk(x2p, NP2), A_2[1], A_2[0], NP2, RPT2, ZB2), N2)
        if i == 1:
            r = conv[:, :4] @ WA_2a + x2p[:, :4] @ WI_2a + b_2a
        else:
            r = conv @ WA_2[i - 2] + x2p @ WI_2[i - 2] + b_2[i - 2]
        r = relu(_bn(r, bn_gamma[7 + i], bn_beta[7 + i]))
        x2p = r if i == 1 else x2p + r
    x2f = relu(_bn(x2p @ W_ge2 + b_ge2, bn_gamma[16], bn_beta[16]))

    s3 = _from_chunk(
        _sc_spmm(_to_chunk(x2f, NP2), A_3, iota1, NP1, RPT1, ZB1), N1)
    a3 = s3 @ W_a3 + b_a3
    a3 = relu(_bn(a3, bn_gamma[17], bn_beta[17]))
    x1 = x1 + a3

    for i in range(1, NUM_LAYERS + 1):
        x1c = _to_chunk(x1, NP1)
        c1 = _from_chunk(_sc_spmm(x1c, E_1[1], E_1[0], NP1, RPT1, ZB1), N1)
        c2 = _from_chunk(_sc_spmm(x1c, E_2[1], E_2[0], NP1, RPT1, ZB1), N1)
        c3 = _from_chunk(_sc_spmm(x1c, E_3[1], E_3[0], NP1, RPT1, ZB1), N1)
        r = (c1 @ W1_1[i - 1] + c2 @ W2_1[i - 1] + c3 @ W3_1[i - 1]
             + x1 @ W0_1[i - 1] + b_1[i - 1])
        r = relu(_bn(r, bn_gamma[i], bn_beta[i]))
        x1 = x1 + r
    x1 = relu(_bn(x1 @ W_ge1 + b_ge1, bn_gamma[15], bn_beta[15]))
    return _softmax_head(x1, W_gef, b_gef)
